# Initial kernel scaffold; baseline (speedup 1.0000x reference)
import functools

import jax
import jax.numpy as jnp
from jax import lax
from jax.experimental import pallas as pl
from jax.experimental.pallas import tpu as pltpu
from jax.experimental.pallas import tpu_sc as plsc

N = 10000
E = 320000
D = 128
NS = 16
CHUNK = 128
NCHUNK = -(-E // (NS * CHUNK))
EPT = NCHUNK * CHUNK
E_PAD = NS * EPT
RPT = 626
N_PAD = NS * RPT

_f32 = jnp.float32


def _agg_body(with_counts, *refs):
    if with_counts:
        (xu, xi, sui, dui, siu, diu, zrow, zcnt, ones_h,
         agg_i, cnt_i, agg_u, cnt_u,
         sidx, didx, rows, ones_v, agg_s, cnt_s, sem) = refs
    else:
        (xu, xi, sui, dui, siu, diu, zrow,
         agg_i, agg_u,
         sidx, didx, rows, agg_s, sem) = refs

    c = lax.axis_index("c")
    s = lax.axis_index("s")
    share = pl.ds(s * RPT, RPT)

    pltpu.sync_copy(zrow, agg_s.at[share])
    if with_counts:
        pltpu.sync_copy(zcnt, cnt_s.at[share])
        pltpu.sync_copy(ones_h, ones_v)
    plsc.subcore_barrier()

    def do_rel(x_src, src_e, dst_e, agg_out, cnt_out):
        pltpu.sync_copy(src_e.at[s], sidx)
        pltpu.sync_copy(dst_e.at[s], didx)

        def chunk_body(i, carry):
            pltpu.async_copy(x_src.at[sidx.at[i]], rows, sem).wait()
            pltpu.sync_copy(rows, agg_s.at[didx.at[i]], add=True)
            if with_counts:
                pltpu.sync_copy(ones_v, cnt_s.at[didx.at[i]], add=True)
            return carry

        lax.fori_loop(0, NCHUNK, chunk_body, 0)
        plsc.subcore_barrier()
        pltpu.sync_copy(agg_s.at[share], agg_out.at[share])
        if with_counts:
            pltpu.sync_copy(cnt_s.at[share], cnt_out.at[share])

    @pl.when(c == 0)
    def _():
        do_rel(xu, sui, dui, agg_i, cnt_i if with_counts else None)

    @pl.when(c == 1)
    def _():
        do_rel(xi, siu, diu, agg_u, cnt_u if with_counts else None)


def _make_agg(with_counts):
    mesh = plsc.VectorSubcoreMesh(core_axis_name="c", subcore_axis_name="s")
    agg_t = jax.ShapeDtypeStruct((N_PAD, D), _f32)
    cnt_t = jax.ShapeDtypeStruct((N_PAD, 16), _f32)
    if with_counts:
        out_type = (agg_t, cnt_t, agg_t, cnt_t)
    else:
        out_type = (agg_t, agg_t)
    scratch = [
        pltpu.VMEM((NCHUNK, CHUNK), jnp.int32),
        pltpu.VMEM((NCHUNK, CHUNK), jnp.int32),
        pltpu.VMEM((CHUNK, D), _f32),
    ]
    if with_counts:
        scratch.append(pltpu.VMEM((CHUNK, 16), _f32))
    scratch.append(pltpu.VMEM_SHARED((N_PAD, D), _f32))
    if with_counts:
        scratch.append(pltpu.VMEM_SHARED((N_PAD, 16), _f32))
    scratch.append(pltpu.SemaphoreType.DMA)
    return pl.kernel(
        functools.partial(_agg_body, with_counts),
        out_type=out_type,
        mesh=mesh,
        scratch_types=scratch,
        name="sage_agg_cnt" if with_counts else "sage_agg",
    )


_agg_with_counts = _make_agg(True)
_agg_no_counts = _make_agg(False)


def _sage_mm_body(relu, agg_i, cnt_i, xi, Wli, bli, Wri,
                  agg_u, cnt_u, xu, Wlu, blu, Wru, hi, hu):
    def one(agg, cnt, x, Wl, b, Wr, out):
        deg = jnp.maximum(cnt[0:N, 0:1], 1.0)
        mean = agg[0:N, :] / deg
        h = (jnp.dot(mean, Wl[...], preferred_element_type=_f32)
             + b[...]
             + jnp.dot(x[...], Wr[...], preferred_element_type=_f32))
        out[...] = jnp.maximum(h, 0.0) if relu else h

    one(agg_i, cnt_i, xi, Wli, bli, Wri, hi)
    one(agg_u, cnt_u, xu, Wlu, blu, Wru, hu)


def _make_mm(relu):
    return pl.pallas_call(
        functools.partial(_sage_mm_body, relu),
        out_shape=(jax.ShapeDtypeStruct((N, D), _f32),
                   jax.ShapeDtypeStruct((N, D), _f32)),
        name="sage_mm_relu" if relu else "sage_mm",
    )


_mm_relu = _make_mm(True)
_mm_lin = _make_mm(False)


def _prep_edges(e):
    pad = E_PAD - E
    src = jnp.concatenate([e[0].astype(jnp.int32), jnp.zeros((pad,), jnp.int32)])
    dst = jnp.concatenate([e[1].astype(jnp.int32), jnp.full((pad,), N, jnp.int32)])
    return (src.reshape(NS, NCHUNK, CHUNK), dst.reshape(NS, NCHUNK, CHUNK))


def kernel(x_user, x_item, edge_index_ui, edge_index_iu,
           W1_ui_l, b1_ui_l, W1_ui_r, W1_iu_l, b1_iu_l, W1_iu_r,
           W2_ui_l, b2_ui_l, W2_ui_r, W2_iu_l, b2_iu_l, W2_iu_r):
    sui, dui = _prep_edges(edge_index_ui)
    siu, diu = _prep_edges(edge_index_iu)
    zrow = jnp.zeros((RPT, D), _f32)
    zcnt = jnp.zeros((RPT, 16), _f32)
    ones_h = jnp.ones((CHUNK, 16), _f32)

    agg1_i, cnt_i, agg1_u, cnt_u = _agg_with_counts(
        x_user, x_item, sui, dui, siu, diu, zrow, zcnt, ones_h)
    h_item, h_user = _mm_relu(
        agg1_i, cnt_i, x_item, W1_ui_l, b1_ui_l.reshape(1, D), W1_ui_r,
        agg1_u, cnt_u, x_user, W1_iu_l, b1_iu_l.reshape(1, D), W1_iu_r)

    agg2_i, agg2_u = _agg_no_counts(
        h_user, h_item, sui, dui, siu, diu, zrow)
    out_item, out_user = _mm_lin(
        agg2_i, cnt_i, h_item, W2_ui_l, b2_ui_l.reshape(1, D), W2_ui_r,
        agg2_u, cnt_u, h_user, W2_iu_l, b2_iu_l.reshape(1, D), W2_iu_r)

    return (out_user, out_item)


# speedup vs baseline: 3.5631x; 3.5631x over previous
import functools

import jax
import jax.numpy as jnp
from jax import lax
from jax.experimental import pallas as pl
from jax.experimental.pallas import tpu as pltpu
from jax.experimental.pallas import tpu_sc as plsc

N = 10000
E = 320000
D = 128
NS = 16
CHUNK = 128
G = 16
NGROUP = 10
NCHUNK = NGROUP * G
EPT = NCHUNK * CHUNK
E_PAD = NS * EPT
RPT = 632
N_PAD = NS * RPT

_f32 = jnp.float32


def _agg_body(xu, xi, sui, dui, siu, diu, zrow,
              agg_i, agg_u,
              sidx, didx, rows, agg_s, sem):
    c = lax.axis_index("c")
    s = lax.axis_index("s")
    share = pl.ds(s * RPT, RPT)

    pltpu.sync_copy(zrow, agg_s.at[share])
    plsc.subcore_barrier()

    def do_rel(x_src, src_e, dst_e, agg_out):
        def group_body(g, carry):
            pltpu.sync_copy(src_e.at[s, pl.ds(g * G, G)], sidx)
            pltpu.sync_copy(dst_e.at[s, pl.ds(g * G, G)], didx)

            def chunk_body(j, carry2):
                pltpu.async_copy(x_src.at[sidx.at[j]], rows, sem).wait()
                pltpu.sync_copy(rows, agg_s.at[didx.at[j]], add=True)
                return carry2

            lax.fori_loop(0, G, chunk_body, 0)
            return carry

        lax.fori_loop(0, NGROUP, group_body, 0)
        plsc.subcore_barrier()
        pltpu.sync_copy(agg_s.at[share], agg_out.at[share])

    @pl.when(c == 0)
    def _():
        do_rel(xu, sui, dui, agg_i)

    @pl.when(c == 1)
    def _():
        do_rel(xi, siu, diu, agg_u)


def _cnt_body(dui, diu, zrow, ones_h,
              cnt_i, cnt_u,
              didx, ones_v, cnt_s, sem):
    del sem
    c = lax.axis_index("c")
    s = lax.axis_index("s")
    share = pl.ds(s * RPT, RPT)

    pltpu.sync_copy(zrow, cnt_s.at[share])
    pltpu.sync_copy(ones_h, ones_v)
    plsc.subcore_barrier()

    def do_rel(dst_e, cnt_out):
        def group_body(g, carry):
            pltpu.sync_copy(dst_e.at[s, pl.ds(g * G, G)], didx)

            def chunk_body(j, carry2):
                pltpu.sync_copy(ones_v, cnt_s.at[didx.at[j]], add=True)
                return carry2

            lax.fori_loop(0, G, chunk_body, 0)
            return carry

        lax.fori_loop(0, NGROUP, group_body, 0)
        plsc.subcore_barrier()
        pltpu.sync_copy(cnt_s.at[share], cnt_out.at[share])

    @pl.when(c == 0)
    def _():
        do_rel(dui, cnt_i)

    @pl.when(c == 1)
    def _():
        do_rel(diu, cnt_u)


def _make_agg():
    mesh = plsc.VectorSubcoreMesh(core_axis_name="c", subcore_axis_name="s")
    agg_t = jax.ShapeDtypeStruct((N_PAD, D), _f32)
    return pl.kernel(
        _agg_body,
        out_type=(agg_t, agg_t),
        mesh=mesh,
        scratch_types=[
            pltpu.VMEM((G, CHUNK), jnp.int32),
            pltpu.VMEM((G, CHUNK), jnp.int32),
            pltpu.VMEM((CHUNK, D), _f32),
            pltpu.VMEM_SHARED((N_PAD, D), _f32),
            pltpu.SemaphoreType.DMA,
        ],
        name="sage_agg",
    )


def _make_cnt():
    mesh = plsc.VectorSubcoreMesh(core_axis_name="c", subcore_axis_name="s")
    cnt_t = jax.ShapeDtypeStruct((N_PAD, D), _f32)
    return pl.kernel(
        _cnt_body,
        out_type=(cnt_t, cnt_t),
        mesh=mesh,
        scratch_types=[
            pltpu.VMEM((G, CHUNK), jnp.int32),
            pltpu.VMEM((CHUNK, D), _f32),
            pltpu.VMEM_SHARED((N_PAD, D), _f32),
            pltpu.SemaphoreType.DMA,
        ],
        name="sage_cnt",
    )


_agg_pass = _make_agg()
_cnt_pass = _make_cnt()


def _sage_mm_body(relu, agg_i, cnt_i, xi, Wli, bli, Wri,
                  agg_u, cnt_u, xu, Wlu, blu, Wru, hi, hu):
    def one(agg, cnt, x, Wl, b, Wr, out):
        deg = jnp.maximum(cnt[0:N, 0:1], 1.0)
        mean = agg[0:N, :] / deg
        h = (jnp.dot(mean, Wl[...], preferred_element_type=_f32)
             + b[...]
             + jnp.dot(x[...], Wr[...], preferred_element_type=_f32))
        out[...] = jnp.maximum(h, 0.0) if relu else h

    one(agg_i, cnt_i, xi, Wli, bli, Wri, hi)
    one(agg_u, cnt_u, xu, Wlu, blu, Wru, hu)


def _make_mm(relu):
    return pl.pallas_call(
        functools.partial(_sage_mm_body, relu),
        out_shape=(jax.ShapeDtypeStruct((N, D), _f32),
                   jax.ShapeDtypeStruct((N, D), _f32)),
        name="sage_mm_relu" if relu else "sage_mm",
    )


_mm_relu = _make_mm(True)
_mm_lin = _make_mm(False)


def _prep_edges(e):
    pad = E_PAD - E
    src = jnp.concatenate([e[0].astype(jnp.int32), jnp.zeros((pad,), jnp.int32)])
    dst = jnp.concatenate([e[1].astype(jnp.int32), jnp.full((pad,), N, jnp.int32)])
    return (src.reshape(NS, NCHUNK, CHUNK), dst.reshape(NS, NCHUNK, CHUNK))


def kernel(x_user, x_item, edge_index_ui, edge_index_iu,
           W1_ui_l, b1_ui_l, W1_ui_r, W1_iu_l, b1_iu_l, W1_iu_r,
           W2_ui_l, b2_ui_l, W2_ui_r, W2_iu_l, b2_iu_l, W2_iu_r):
    sui, dui = _prep_edges(edge_index_ui)
    siu, diu = _prep_edges(edge_index_iu)
    zrow = jnp.zeros((RPT, D), _f32)
    ones_h = jnp.ones((CHUNK, D), _f32)

    cnt_i, cnt_u = _cnt_pass(dui, diu, zrow, ones_h)

    agg1_i, agg1_u = _agg_pass(x_user, x_item, sui, dui, siu, diu, zrow)
    h_item, h_user = _mm_relu(
        agg1_i, cnt_i, x_item, W1_ui_l, b1_ui_l.reshape(1, D), W1_ui_r,
        agg1_u, cnt_u, x_user, W1_iu_l, b1_iu_l.reshape(1, D), W1_iu_r)

    agg2_i, agg2_u = _agg_pass(h_user, h_item, sui, dui, siu, diu, zrow)
    out_item, out_user = _mm_lin(
        agg2_i, cnt_i, h_item, W2_ui_l, b2_ui_l.reshape(1, D), W2_ui_r,
        agg2_u, cnt_u, h_user, W2_iu_l, b2_iu_l.reshape(1, D), W2_iu_r)

    return (out_user, out_item)


# speedup vs baseline: 3.9703x; 1.1143x over previous
import functools

import jax
import jax.numpy as jnp
from jax import lax
from jax.experimental import pallas as pl
from jax.experimental.pallas import tpu as pltpu
from jax.experimental.pallas import tpu_sc as plsc

N = 10000
E = 320000
D = 128
NS = 16
CHUNK = 128
G = 16
NGROUP = 10
NCHUNK = NGROUP * G
EPT = NCHUNK * CHUNK
E_PAD = NS * EPT
RPT = 632
N_PAD = NS * RPT

_f32 = jnp.float32


def _agg_body(xu, xi, sui, dui, siu, diu, zrow,
              agg_i, agg_u,
              sidx, didx, rows0, rows1, agg_s, gsem0, gsem1, ssem0, ssem1):
    c = lax.axis_index("c")
    s = lax.axis_index("s")
    share = pl.ds(s * RPT, RPT)

    pltpu.sync_copy(zrow, agg_s.at[share])
    plsc.subcore_barrier()

    rowbufs = (rows0, rows1)
    gsems = (gsem0, gsem1)
    ssems = (ssem0, ssem1)

    def do_rel(x_src, src_e, dst_e, agg_out):
        def group_body(g, carry):
            pltpu.sync_copy(src_e.at[s, pl.ds(g * G, G)], sidx)
            pltpu.sync_copy(dst_e.at[s, pl.ds(g * G, G)], didx)
            gd = [None] * G
            sd = [None] * G
            gd[0] = pltpu.async_copy(x_src.at[sidx.at[0]], rows0, gsem0)
            for jj in range(G):
                p = jj & 1
                gd[jj].wait()
                sd[jj] = pltpu.async_copy(
                    rowbufs[p], agg_s.at[didx.at[jj]], ssems[p], add=True)
                if jj + 1 < G:
                    if jj >= 1:
                        sd[jj - 1].wait()
                    gd[jj + 1] = pltpu.async_copy(
                        x_src.at[sidx.at[jj + 1]], rowbufs[1 - p], gsems[1 - p])
            sd[G - 2].wait()
            sd[G - 1].wait()
            return carry

        lax.fori_loop(0, NGROUP, group_body, 0)
        plsc.subcore_barrier()
        pltpu.sync_copy(agg_s.at[share], agg_out.at[share])

    @pl.when(c == 0)
    def _():
        do_rel(xu, sui, dui, agg_i)

    @pl.when(c == 1)
    def _():
        do_rel(xi, siu, diu, agg_u)


def _cnt_body(dui, diu, zrow, ones_h,
              cnt_i, cnt_u,
              didx, ones_v, cnt_s, sem):
    c = lax.axis_index("c")
    s = lax.axis_index("s")
    share = pl.ds(s * RPT, RPT)

    pltpu.sync_copy(zrow, cnt_s.at[share])
    pltpu.sync_copy(ones_h, ones_v)
    plsc.subcore_barrier()

    def do_rel(dst_e, cnt_out):
        def group_body(g, carry):
            pltpu.sync_copy(dst_e.at[s, pl.ds(g * G, G)], didx)
            descs = [
                pltpu.async_copy(ones_v, cnt_s.at[didx.at[j]], sem, add=True)
                for j in range(G)
            ]
            for d in descs:
                d.wait()
            return carry

        lax.fori_loop(0, NGROUP, group_body, 0)
        plsc.subcore_barrier()
        pltpu.sync_copy(cnt_s.at[share], cnt_out.at[share])

    @pl.when(c == 0)
    def _():
        do_rel(dui, cnt_i)

    @pl.when(c == 1)
    def _():
        do_rel(diu, cnt_u)


def _make_agg():
    mesh = plsc.VectorSubcoreMesh(core_axis_name="c", subcore_axis_name="s")
    agg_t = jax.ShapeDtypeStruct((N_PAD, D), _f32)
    return pl.kernel(
        _agg_body,
        out_type=(agg_t, agg_t),
        mesh=mesh,
        scratch_types=[
            pltpu.VMEM((G, CHUNK), jnp.int32),
            pltpu.VMEM((G, CHUNK), jnp.int32),
            pltpu.VMEM((CHUNK, D), _f32),
            pltpu.VMEM((CHUNK, D), _f32),
            pltpu.VMEM_SHARED((N_PAD, D), _f32),
            pltpu.SemaphoreType.DMA,
            pltpu.SemaphoreType.DMA,
            pltpu.SemaphoreType.DMA,
            pltpu.SemaphoreType.DMA,
        ],
        name="sage_agg",
    )


def _make_cnt():
    mesh = plsc.VectorSubcoreMesh(core_axis_name="c", subcore_axis_name="s")
    cnt_t = jax.ShapeDtypeStruct((N_PAD, D), _f32)
    return pl.kernel(
        _cnt_body,
        out_type=(cnt_t, cnt_t),
        mesh=mesh,
        scratch_types=[
            pltpu.VMEM((G, CHUNK), jnp.int32),
            pltpu.VMEM((CHUNK, D), _f32),
            pltpu.VMEM_SHARED((N_PAD, D), _f32),
            pltpu.SemaphoreType.DMA,
        ],
        name="sage_cnt",
    )


_agg_pass = _make_agg()
_cnt_pass = _make_cnt()


def _sage_mm_body(relu, agg_i, cnt_i, xi, Wli, bli, Wri,
                  agg_u, cnt_u, xu, Wlu, blu, Wru, hi, hu):
    def one(agg, cnt, x, Wl, b, Wr, out):
        deg = jnp.maximum(cnt[0:N, 0:1], 1.0)
        mean = agg[0:N, :] / deg
        h = (jnp.dot(mean, Wl[...], preferred_element_type=_f32)
             + b[...]
             + jnp.dot(x[...], Wr[...], preferred_element_type=_f32))
        out[...] = jnp.maximum(h, 0.0) if relu else h

    one(agg_i, cnt_i, xi, Wli, bli, Wri, hi)
    one(agg_u, cnt_u, xu, Wlu, blu, Wru, hu)


def _make_mm(relu):
    return pl.pallas_call(
        functools.partial(_sage_mm_body, relu),
        out_shape=(jax.ShapeDtypeStruct((N, D), _f32),
                   jax.ShapeDtypeStruct((N, D), _f32)),
        name="sage_mm_relu" if relu else "sage_mm",
    )


_mm_relu = _make_mm(True)
_mm_lin = _make_mm(False)


def _prep_edges(e):
    pad = E_PAD - E
    src = jnp.concatenate([e[0].astype(jnp.int32), jnp.zeros((pad,), jnp.int32)])
    dst = jnp.concatenate([e[1].astype(jnp.int32), jnp.full((pad,), N, jnp.int32)])
    return (src.reshape(NS, NCHUNK, CHUNK), dst.reshape(NS, NCHUNK, CHUNK))


def kernel(x_user, x_item, edge_index_ui, edge_index_iu,
           W1_ui_l, b1_ui_l, W1_ui_r, W1_iu_l, b1_iu_l, W1_iu_r,
           W2_ui_l, b2_ui_l, W2_ui_r, W2_iu_l, b2_iu_l, W2_iu_r):
    sui, dui = _prep_edges(edge_index_ui)
    siu, diu = _prep_edges(edge_index_iu)
    zrow = jnp.zeros((RPT, D), _f32)
    ones_h = jnp.ones((CHUNK, D), _f32)

    cnt_i, cnt_u = _cnt_pass(dui, diu, zrow, ones_h)

    agg1_i, agg1_u = _agg_pass(x_user, x_item, sui, dui, siu, diu, zrow)
    h_item, h_user = _mm_relu(
        agg1_i, cnt_i, x_item, W1_ui_l, b1_ui_l.reshape(1, D), W1_ui_r,
        agg1_u, cnt_u, x_user, W1_iu_l, b1_iu_l.reshape(1, D), W1_iu_r)

    agg2_i, agg2_u = _agg_pass(h_user, h_item, sui, dui, siu, diu, zrow)
    out_item, out_user = _mm_lin(
        agg2_i, cnt_i, h_item, W2_ui_l, b2_ui_l.reshape(1, D), W2_ui_r,
        agg2_u, cnt_u, h_user, W2_iu_l, b2_iu_l.reshape(1, D), W2_iu_r)

    return (out_user, out_item)


# speedup vs baseline: 4.3235x; 1.0890x over previous
import functools

import jax
import jax.numpy as jnp
from jax import lax
from jax.experimental import pallas as pl
from jax.experimental.pallas import tpu as pltpu
from jax.experimental.pallas import tpu_sc as plsc

N = 10000
E = 320000
D = 128
NS = 16
EPT = 20480
E_PAD = NS * EPT
RPT = 632
N_PAD = NS * RPT

A_CHUNK = 64
A_G = 32
A_NGROUP = EPT // (A_G * A_CHUNK)
NBUF = 4

C_CHUNK = 128
C_G = 16
C_NGROUP = EPT // (C_G * C_CHUNK)

_f32 = jnp.float32


def _agg_body(xu, xi, sui, dui, siu, diu, zrow,
              agg_i, agg_u,
              sidx, didx, *rest):
    rowbufs = rest[:NBUF]
    agg_s = rest[NBUF]
    gsems = rest[NBUF + 1:2 * NBUF + 1]
    ssems = rest[2 * NBUF + 1:3 * NBUF + 1]

    c = lax.axis_index("c")
    s = lax.axis_index("s")
    share = pl.ds(s * RPT, RPT)

    pltpu.sync_copy(zrow, agg_s.at[share])
    plsc.subcore_barrier()

    def do_rel(x_src, src_e, dst_e, agg_out):
        def group_body(g, carry):
            pltpu.sync_copy(src_e.at[s, pl.ds(g * A_G, A_G)], sidx)
            pltpu.sync_copy(dst_e.at[s, pl.ds(g * A_G, A_G)], didx)
            gd = [None] * A_G
            sd = [None] * A_G
            for jj in range(NBUF - 1):
                gd[jj] = pltpu.async_copy(
                    x_src.at[sidx.at[jj]], rowbufs[jj % NBUF], gsems[jj % NBUF])
            for jj in range(A_G):
                p = jj % NBUF
                gd[jj].wait()
                sd[jj] = pltpu.async_copy(
                    rowbufs[p], agg_s.at[didx.at[jj]], ssems[p], add=True)
                nxt = jj + NBUF - 1
                if nxt < A_G:
                    q = nxt % NBUF
                    if nxt - NBUF >= 0:
                        sd[nxt - NBUF].wait()
                    gd[nxt] = pltpu.async_copy(
                        x_src.at[sidx.at[nxt]], rowbufs[q], gsems[q])
            for k in range(max(0, A_G - NBUF), A_G):
                sd[k].wait()
            return carry

        lax.fori_loop(0, A_NGROUP, group_body, 0)
        plsc.subcore_barrier()
        pltpu.sync_copy(agg_s.at[share], agg_out.at[share])

    @pl.when(c == 0)
    def _():
        do_rel(xu, sui, dui, agg_i)

    @pl.when(c == 1)
    def _():
        do_rel(xi, siu, diu, agg_u)


def _cnt_body(dui, diu, zrow, ones_h,
              cnt_i, cnt_u,
              didx, ones_v, cnt_s, sem):
    c = lax.axis_index("c")
    s = lax.axis_index("s")
    share = pl.ds(s * RPT, RPT)

    pltpu.sync_copy(zrow, cnt_s.at[share])
    pltpu.sync_copy(ones_h, ones_v)
    plsc.subcore_barrier()

    def do_rel(dst_e, cnt_out):
        def group_body(g, carry):
            pltpu.sync_copy(dst_e.at[s, pl.ds(g * C_G, C_G)], didx)
            descs = [
                pltpu.async_copy(ones_v, cnt_s.at[didx.at[j]], sem, add=True)
                for j in range(C_G)
            ]
            for d in descs:
                d.wait()
            return carry

        lax.fori_loop(0, C_NGROUP, group_body, 0)
        plsc.subcore_barrier()
        pltpu.sync_copy(cnt_s.at[share], cnt_out.at[share])

    @pl.when(c == 0)
    def _():
        do_rel(dui, cnt_i)

    @pl.when(c == 1)
    def _():
        do_rel(diu, cnt_u)


def _make_agg():
    mesh = plsc.VectorSubcoreMesh(core_axis_name="c", subcore_axis_name="s")
    agg_t = jax.ShapeDtypeStruct((N_PAD, D), _f32)
    scratch = [
        pltpu.VMEM((A_G, A_CHUNK), jnp.int32),
        pltpu.VMEM((A_G, A_CHUNK), jnp.int32),
    ]
    scratch += [pltpu.VMEM((A_CHUNK, D), _f32) for _ in range(NBUF)]
    scratch.append(pltpu.VMEM_SHARED((N_PAD, D), _f32))
    scratch += [pltpu.SemaphoreType.DMA for _ in range(2 * NBUF)]
    return pl.kernel(
        _agg_body,
        out_type=(agg_t, agg_t),
        mesh=mesh,
        scratch_types=scratch,
        name="sage_agg",
    )


def _make_cnt():
    mesh = plsc.VectorSubcoreMesh(core_axis_name="c", subcore_axis_name="s")
    cnt_t = jax.ShapeDtypeStruct((N_PAD, D), _f32)
    return pl.kernel(
        _cnt_body,
        out_type=(cnt_t, cnt_t),
        mesh=mesh,
        scratch_types=[
            pltpu.VMEM((C_G, C_CHUNK), jnp.int32),
            pltpu.VMEM((C_CHUNK, D), _f32),
            pltpu.VMEM_SHARED((N_PAD, D), _f32),
            pltpu.SemaphoreType.DMA,
        ],
        name="sage_cnt",
    )


_agg_pass = _make_agg()
_cnt_pass = _make_cnt()


def _sage_mm_body(relu, agg_i, cnt_i, xi, Wli, bli, Wri,
                  agg_u, cnt_u, xu, Wlu, blu, Wru, hi, hu):
    def one(agg, cnt, x, Wl, b, Wr, out):
        deg = jnp.maximum(cnt[0:N, 0:1], 1.0)
        mean = agg[0:N, :] / deg
        h = (jnp.dot(mean, Wl[...], preferred_element_type=_f32)
             + b[...]
             + jnp.dot(x[...], Wr[...], preferred_element_type=_f32))
        out[...] = jnp.maximum(h, 0.0) if relu else h

    one(agg_i, cnt_i, xi, Wli, bli, Wri, hi)
    one(agg_u, cnt_u, xu, Wlu, blu, Wru, hu)


def _make_mm(relu):
    return pl.pallas_call(
        functools.partial(_sage_mm_body, relu),
        out_shape=(jax.ShapeDtypeStruct((N, D), _f32),
                   jax.ShapeDtypeStruct((N, D), _f32)),
        name="sage_mm_relu" if relu else "sage_mm",
    )


_mm_relu = _make_mm(True)
_mm_lin = _make_mm(False)


def _prep_edges(e):
    pad = E_PAD - E
    src = jnp.concatenate([e[0].astype(jnp.int32), jnp.zeros((pad,), jnp.int32)])
    dst = jnp.concatenate([e[1].astype(jnp.int32), jnp.full((pad,), N, jnp.int32)])
    sa = src.reshape(NS, A_G * A_NGROUP, A_CHUNK)
    da = dst.reshape(NS, A_G * A_NGROUP, A_CHUNK)
    dc = dst.reshape(NS, C_G * C_NGROUP, C_CHUNK)
    return sa, da, dc


def kernel(x_user, x_item, edge_index_ui, edge_index_iu,
           W1_ui_l, b1_ui_l, W1_ui_r, W1_iu_l, b1_iu_l, W1_iu_r,
           W2_ui_l, b2_ui_l, W2_ui_r, W2_iu_l, b2_iu_l, W2_iu_r):
    sui, dui, dui_c = _prep_edges(edge_index_ui)
    siu, diu, diu_c = _prep_edges(edge_index_iu)
    zrow = jnp.zeros((RPT, D), _f32)
    ones_h = jnp.ones((C_CHUNK, D), _f32)

    cnt_i, cnt_u = _cnt_pass(dui_c, diu_c, zrow, ones_h)

    agg1_i, agg1_u = _agg_pass(x_user, x_item, sui, dui, siu, diu, zrow)
    h_item, h_user = _mm_relu(
        agg1_i, cnt_i, x_item, W1_ui_l, b1_ui_l.reshape(1, D), W1_ui_r,
        agg1_u, cnt_u, x_user, W1_iu_l, b1_iu_l.reshape(1, D), W1_iu_r)

    agg2_i, agg2_u = _agg_pass(h_user, h_item, sui, dui, siu, diu, zrow)
    out_item, out_user = _mm_lin(
        agg2_i, cnt_i, h_item, W2_ui_l, b2_ui_l.reshape(1, D), W2_ui_r,
        agg2_u, cnt_u, h_user, W2_iu_l, b2_iu_l.reshape(1, D), W2_iu_r)

    return (out_user, out_item)


# speedup vs baseline: 4.3902x; 1.0154x over previous
import functools

import jax
import jax.numpy as jnp
from jax import lax
from jax.experimental import pallas as pl
from jax.experimental.pallas import tpu as pltpu
from jax.experimental.pallas import tpu_sc as plsc

N = 10000
E = 320000
D = 128
NS = 16
EPT = 20480
E_PAD = NS * EPT
RPT = 632
N_PAD = NS * RPT

A_CHUNK = 64
A_G = 32
A_NGROUP = EPT // (A_G * A_CHUNK)
NBUF = 4

C_CHUNK = 128
C_G = 16
C_NGROUP = EPT // (C_G * C_CHUNK)

_f32 = jnp.float32


def _agg_body(xu, xi, sui, dui, siu, diu, zrow,
              agg_i, agg_u,
              sidx, didx, *rest):
    rowbufs = rest[:NBUF]
    agg_s = rest[NBUF]
    gsems = rest[NBUF + 1:2 * NBUF + 1]
    ssems = rest[2 * NBUF + 1:3 * NBUF + 1]
    isem0 = rest[3 * NBUF + 1]
    isem1 = rest[3 * NBUF + 2]

    c = lax.axis_index("c")
    s = lax.axis_index("s")
    share = pl.ds(s * RPT, RPT)

    pltpu.sync_copy(zrow, agg_s.at[share])
    plsc.subcore_barrier()

    def do_rel(x_src, src_e, dst_e, agg_out):
        pltpu.async_copy(src_e.at[s, pl.ds(0, A_G)], sidx.at[0], isem0)
        pltpu.async_copy(dst_e.at[s, pl.ds(0, A_G)], didx.at[0], isem1)

        def group_body(g, carry):
            par = lax.rem(g, 2)
            sg = sidx.at[par]
            dg = didx.at[par]
            pltpu.make_async_copy(
                src_e.at[s, pl.ds(g * A_G, A_G)], sg, isem0).wait()
            pltpu.make_async_copy(
                dst_e.at[s, pl.ds(g * A_G, A_G)], dg, isem1).wait()
            gn = lax.min(g + 1, A_NGROUP - 1)
            pltpu.async_copy(
                src_e.at[s, pl.ds(gn * A_G, A_G)], sidx.at[1 - par], isem0)
            pltpu.async_copy(
                dst_e.at[s, pl.ds(gn * A_G, A_G)], didx.at[1 - par], isem1)

            gd = [None] * A_G
            sd = [None] * A_G
            for jj in range(NBUF - 1):
                gd[jj] = pltpu.async_copy(
                    x_src.at[sg.at[jj]], rowbufs[jj % NBUF], gsems[jj % NBUF])
            for jj in range(A_G):
                p = jj % NBUF
                gd[jj].wait()
                sd[jj] = pltpu.async_copy(
                    rowbufs[p], agg_s.at[dg.at[jj]], ssems[p], add=True)
                nxt = jj + NBUF - 1
                if nxt < A_G:
                    q = nxt % NBUF
                    if nxt - NBUF >= 0:
                        sd[nxt - NBUF].wait()
                    gd[nxt] = pltpu.async_copy(
                        x_src.at[sg.at[nxt]], rowbufs[q], gsems[q])
            for k in range(max(0, A_G - NBUF), A_G):
                sd[k].wait()
            return carry

        lax.fori_loop(0, A_NGROUP, group_body, 0)
        lastbuf = A_NGROUP % 2
        pltpu.make_async_copy(
            src_e.at[s, pl.ds(0, A_G)], sidx.at[lastbuf], isem0).wait()
        pltpu.make_async_copy(
            dst_e.at[s, pl.ds(0, A_G)], didx.at[lastbuf], isem1).wait()
        plsc.subcore_barrier()
        pltpu.sync_copy(agg_s.at[share], agg_out.at[share])

    @pl.when(c == 0)
    def _():
        do_rel(xu, sui, dui, agg_i)

    @pl.when(c == 1)
    def _():
        do_rel(xi, siu, diu, agg_u)


def _cnt_body(dui, diu, zrow, ones_h,
              cnt_i, cnt_u,
              didx, ones_v, cnt_s, sem):
    c = lax.axis_index("c")
    s = lax.axis_index("s")
    share = pl.ds(s * RPT, RPT)

    pltpu.sync_copy(zrow, cnt_s.at[share])
    pltpu.sync_copy(ones_h, ones_v)
    plsc.subcore_barrier()

    def do_rel(dst_e, cnt_out):
        def group_body(g, carry):
            pltpu.sync_copy(dst_e.at[s, pl.ds(g * C_G, C_G)], didx)
            descs = [
                pltpu.async_copy(ones_v, cnt_s.at[didx.at[j]], sem, add=True)
                for j in range(C_G)
            ]
            for d in descs:
                d.wait()
            return carry

        lax.fori_loop(0, C_NGROUP, group_body, 0)
        plsc.subcore_barrier()
        pltpu.sync_copy(cnt_s.at[share], cnt_out.at[share])

    @pl.when(c == 0)
    def _():
        do_rel(dui, cnt_i)

    @pl.when(c == 1)
    def _():
        do_rel(diu, cnt_u)


def _make_agg():
    mesh = plsc.VectorSubcoreMesh(core_axis_name="c", subcore_axis_name="s")
    agg_t = jax.ShapeDtypeStruct((N_PAD, D), _f32)
    scratch = [
        pltpu.VMEM((2, A_G, A_CHUNK), jnp.int32),
        pltpu.VMEM((2, A_G, A_CHUNK), jnp.int32),
    ]
    scratch += [pltpu.VMEM((A_CHUNK, D), _f32) for _ in range(NBUF)]
    scratch.append(pltpu.VMEM_SHARED((N_PAD, D), _f32))
    scratch += [pltpu.SemaphoreType.DMA for _ in range(2 * NBUF + 2)]
    return pl.kernel(
        _agg_body,
        out_type=(agg_t, agg_t),
        mesh=mesh,
        scratch_types=scratch,
        name="sage_agg",
    )


def _make_cnt():
    mesh = plsc.VectorSubcoreMesh(core_axis_name="c", subcore_axis_name="s")
    cnt_t = jax.ShapeDtypeStruct((N_PAD, D), _f32)
    return pl.kernel(
        _cnt_body,
        out_type=(cnt_t, cnt_t),
        mesh=mesh,
        scratch_types=[
            pltpu.VMEM((C_G, C_CHUNK), jnp.int32),
            pltpu.VMEM((C_CHUNK, D), _f32),
            pltpu.VMEM_SHARED((N_PAD, D), _f32),
            pltpu.SemaphoreType.DMA,
        ],
        name="sage_cnt",
    )


_agg_pass = _make_agg()
_cnt_pass = _make_cnt()


def _sage_mm_body(relu, agg_i, cnt_i, xi, Wli, bli, Wri,
                  agg_u, cnt_u, xu, Wlu, blu, Wru, hi, hu):
    def one(agg, cnt, x, Wl, b, Wr, out):
        deg = jnp.maximum(cnt[0:N, 0:1], 1.0)
        mean = agg[0:N, :] / deg
        h = (jnp.dot(mean, Wl[...], preferred_element_type=_f32)
             + b[...]
             + jnp.dot(x[...], Wr[...], preferred_element_type=_f32))
        out[...] = jnp.maximum(h, 0.0) if relu else h

    one(agg_i, cnt_i, xi, Wli, bli, Wri, hi)
    one(agg_u, cnt_u, xu, Wlu, blu, Wru, hu)


def _make_mm(relu):
    return pl.pallas_call(
        functools.partial(_sage_mm_body, relu),
        out_shape=(jax.ShapeDtypeStruct((N, D), _f32),
                   jax.ShapeDtypeStruct((N, D), _f32)),
        name="sage_mm_relu" if relu else "sage_mm",
    )


_mm_relu = _make_mm(True)
_mm_lin = _make_mm(False)


def _prep_edges(e):
    pad = E_PAD - E
    src = jnp.concatenate([e[0].astype(jnp.int32), jnp.zeros((pad,), jnp.int32)])
    dst = jnp.concatenate([e[1].astype(jnp.int32), jnp.full((pad,), N, jnp.int32)])
    sa = src.reshape(NS, A_G * A_NGROUP, A_CHUNK)
    da = dst.reshape(NS, A_G * A_NGROUP, A_CHUNK)
    dc = dst.reshape(NS, C_G * C_NGROUP, C_CHUNK)
    return sa, da, dc


def kernel(x_user, x_item, edge_index_ui, edge_index_iu,
           W1_ui_l, b1_ui_l, W1_ui_r, W1_iu_l, b1_iu_l, W1_iu_r,
           W2_ui_l, b2_ui_l, W2_ui_r, W2_iu_l, b2_iu_l, W2_iu_r):
    sui, dui, dui_c = _prep_edges(edge_index_ui)
    siu, diu, diu_c = _prep_edges(edge_index_iu)
    zrow = jnp.zeros((RPT, D), _f32)
    ones_h = jnp.ones((C_CHUNK, D), _f32)

    cnt_i, cnt_u = _cnt_pass(dui_c, diu_c, zrow, ones_h)

    agg1_i, agg1_u = _agg_pass(x_user, x_item, sui, dui, siu, diu, zrow)
    h_item, h_user = _mm_relu(
        agg1_i, cnt_i, x_item, W1_ui_l, b1_ui_l.reshape(1, D), W1_ui_r,
        agg1_u, cnt_u, x_user, W1_iu_l, b1_iu_l.reshape(1, D), W1_iu_r)

    agg2_i, agg2_u = _agg_pass(h_user, h_item, sui, dui, siu, diu, zrow)
    out_item, out_user = _mm_lin(
        agg2_i, cnt_i, h_item, W2_ui_l, b2_ui_l.reshape(1, D), W2_ui_r,
        agg2_u, cnt_u, h_user, W2_iu_l, b2_iu_l.reshape(1, D), W2_iu_r)

    return (out_user, out_item)


# speedup vs baseline: 9.3680x; 2.1339x over previous
import functools

import jax
import jax.numpy as jnp
from jax import lax
from jax.experimental import pallas as pl
from jax.experimental.pallas import tpu as pltpu
from jax.experimental.pallas import tpu_sc as plsc

N = 10000
E = 320000
D = 128
NS = 16
EPT = 20480
E_PAD = NS * EPT
RPT = 632
N_PAD = NS * RPT

A_CHUNK = 64
A_G = 32
A_NGROUP = EPT // (A_G * A_CHUNK)
NBUF = 4

C_CHUNK = 128
C_G = 16
C_NGROUP = EPT // (C_G * C_CHUNK)

_f32 = jnp.float32


def _agg_body(xu, xi, sui, dui, siu, diu, zrow,
              agg_i, agg_u,
              sidx, didx, *rest):
    rowbufs = rest[:NBUF]
    agg_s = rest[NBUF]
    gsems = rest[NBUF + 1:2 * NBUF + 1]
    ssems = rest[2 * NBUF + 1:3 * NBUF + 1]
    isem0 = rest[3 * NBUF + 1]
    isem1 = rest[3 * NBUF + 2]

    c = lax.axis_index("c")
    s = lax.axis_index("s")
    share = pl.ds(s * RPT, RPT)

    pltpu.sync_copy(zrow, agg_s.at[share])
    plsc.subcore_barrier()

    def do_rel(x_src, src_e, dst_e, agg_out):
        pltpu.async_copy(src_e.at[s, pl.ds(0, A_G)], sidx.at[0], isem0)
        pltpu.async_copy(dst_e.at[s, pl.ds(0, A_G)], didx.at[0], isem1)

        def group_body(g, carry):
            par = lax.rem(g, 2)
            sg = sidx.at[par]
            dg = didx.at[par]
            pltpu.make_async_copy(
                src_e.at[s, pl.ds(g * A_G, A_G)], sg, isem0).wait()
            pltpu.make_async_copy(
                dst_e.at[s, pl.ds(g * A_G, A_G)], dg, isem1).wait()
            gn = lax.min(g + 1, A_NGROUP - 1)
            pltpu.async_copy(
                src_e.at[s, pl.ds(gn * A_G, A_G)], sidx.at[1 - par], isem0)
            pltpu.async_copy(
                dst_e.at[s, pl.ds(gn * A_G, A_G)], didx.at[1 - par], isem1)

            gd = [None] * A_G
            sd = [None] * A_G
            for jj in range(NBUF - 1):
                gd[jj] = pltpu.async_copy(
                    x_src.at[sg.at[jj]], rowbufs[jj % NBUF], gsems[jj % NBUF])
            for jj in range(A_G):
                p = jj % NBUF
                gd[jj].wait()
                sd[jj] = pltpu.async_copy(
                    rowbufs[p], agg_s.at[dg.at[jj]], ssems[p], add=True)
                nxt = jj + NBUF - 1
                if nxt < A_G:
                    q = nxt % NBUF
                    if nxt - NBUF >= 0:
                        sd[nxt - NBUF].wait()
                    gd[nxt] = pltpu.async_copy(
                        x_src.at[sg.at[nxt]], rowbufs[q], gsems[q])
            for k in range(max(0, A_G - NBUF), A_G):
                sd[k].wait()
            return carry

        lax.fori_loop(0, A_NGROUP, group_body, 0)
        lastbuf = A_NGROUP % 2
        pltpu.make_async_copy(
            src_e.at[s, pl.ds(0, A_G)], sidx.at[lastbuf], isem0).wait()
        pltpu.make_async_copy(
            dst_e.at[s, pl.ds(0, A_G)], didx.at[lastbuf], isem1).wait()
        plsc.subcore_barrier()
        pltpu.sync_copy(agg_s.at[share], agg_out.at[share])

    @pl.when(c == 0)
    def _():
        do_rel(xu, sui, dui, agg_i)

    @pl.when(c == 1)
    def _():
        do_rel(xi, siu, diu, agg_u)


def _cnt_body(dui, diu, zrow, ones_h,
              cnt_i, cnt_u,
              didx, ones_v, cnt_s, sem):
    c = lax.axis_index("c")
    s = lax.axis_index("s")
    share = pl.ds(s * RPT, RPT)

    pltpu.sync_copy(zrow, cnt_s.at[share])
    pltpu.sync_copy(ones_h, ones_v)
    plsc.subcore_barrier()

    def do_rel(dst_e, cnt_out):
        def group_body(g, carry):
            pltpu.sync_copy(dst_e.at[s, pl.ds(g * C_G, C_G)], didx)
            descs = [
                pltpu.async_copy(ones_v, cnt_s.at[didx.at[j]], sem, add=True)
                for j in range(C_G)
            ]
            for d in descs:
                d.wait()
            return carry

        lax.fori_loop(0, C_NGROUP, group_body, 0)
        plsc.subcore_barrier()
        pltpu.sync_copy(cnt_s.at[share], cnt_out.at[share])

    @pl.when(c == 0)
    def _():
        do_rel(dui, cnt_i)

    @pl.when(c == 1)
    def _():
        do_rel(diu, cnt_u)


def _make_agg():
    mesh = plsc.VectorSubcoreMesh(core_axis_name="c", subcore_axis_name="s")
    agg_t = jax.ShapeDtypeStruct((N_PAD, D), _f32)
    scratch = [
        pltpu.VMEM((2, A_G, A_CHUNK), jnp.int32),
        pltpu.VMEM((2, A_G, A_CHUNK), jnp.int32),
    ]
    scratch += [pltpu.VMEM((A_CHUNK, D), _f32) for _ in range(NBUF)]
    scratch.append(pltpu.VMEM_SHARED((N_PAD, D), _f32))
    scratch += [pltpu.SemaphoreType.DMA for _ in range(2 * NBUF + 2)]
    return pl.kernel(
        _agg_body,
        out_type=(agg_t, agg_t),
        mesh=mesh,
        scratch_types=scratch,
        name="sage_agg",
    )


def _make_cnt():
    mesh = plsc.VectorSubcoreMesh(core_axis_name="c", subcore_axis_name="s")
    cnt_t = jax.ShapeDtypeStruct((N_PAD, D), _f32)
    return pl.kernel(
        _cnt_body,
        out_type=(cnt_t, cnt_t),
        mesh=mesh,
        scratch_types=[
            pltpu.VMEM((C_G, C_CHUNK), jnp.int32),
            pltpu.VMEM((C_CHUNK, D), _f32),
            pltpu.VMEM_SHARED((N_PAD, D), _f32),
            pltpu.SemaphoreType.DMA,
        ],
        name="sage_cnt",
    )


_agg_pass = _make_agg()
_cnt_pass = _make_cnt()


def _sage_mm_body(relu, agg_i, cnt_i, xi, Wli, bli, Wri,
                  agg_u, cnt_u, xu, Wlu, blu, Wru, hi, hu):
    def one(agg, cnt, x, Wl, b, Wr, out):
        deg = jnp.maximum(cnt[0:N, 0:1], 1.0)
        mean = agg[0:N, :] / deg
        h = (jnp.dot(mean, Wl[...], preferred_element_type=_f32)
             + b[...]
             + jnp.dot(x[...], Wr[...], preferred_element_type=_f32))
        out[...] = jnp.maximum(h, 0.0) if relu else h

    one(agg_i, cnt_i, xi, Wli, bli, Wri, hi)
    one(agg_u, cnt_u, xu, Wlu, blu, Wru, hu)


def _make_mm(relu):
    return pl.pallas_call(
        functools.partial(_sage_mm_body, relu),
        out_shape=(jax.ShapeDtypeStruct((N, D), _f32),
                   jax.ShapeDtypeStruct((N, D), _f32)),
        name="sage_mm_relu" if relu else "sage_mm",
    )


_mm_relu = _make_mm(True)
_mm_lin = _make_mm(False)


def _prep_edges(e):
    pad = E_PAD - E
    pad_src = jnp.arange(pad, dtype=jnp.int32) % N
    pad_dst = N + (jnp.arange(pad, dtype=jnp.int32) % (N_PAD - N))
    src = jnp.concatenate([e[0].astype(jnp.int32), pad_src])
    dst = jnp.concatenate([e[1].astype(jnp.int32), pad_dst])
    sa = src.reshape(NS, A_G * A_NGROUP, A_CHUNK)
    da = dst.reshape(NS, A_G * A_NGROUP, A_CHUNK)
    dc = dst.reshape(NS, C_G * C_NGROUP, C_CHUNK)
    return sa, da, dc


def kernel(x_user, x_item, edge_index_ui, edge_index_iu,
           W1_ui_l, b1_ui_l, W1_ui_r, W1_iu_l, b1_iu_l, W1_iu_r,
           W2_ui_l, b2_ui_l, W2_ui_r, W2_iu_l, b2_iu_l, W2_iu_r):
    sui, dui, dui_c = _prep_edges(edge_index_ui)
    siu, diu, diu_c = _prep_edges(edge_index_iu)
    zrow = jnp.zeros((RPT, D), _f32)
    ones_h = jnp.ones((C_CHUNK, D), _f32)

    cnt_i, cnt_u = _cnt_pass(dui_c, diu_c, zrow, ones_h)

    agg1_i, agg1_u = _agg_pass(x_user, x_item, sui, dui, siu, diu, zrow)
    h_item, h_user = _mm_relu(
        agg1_i, cnt_i, x_item, W1_ui_l, b1_ui_l.reshape(1, D), W1_ui_r,
        agg1_u, cnt_u, x_user, W1_iu_l, b1_iu_l.reshape(1, D), W1_iu_r)

    agg2_i, agg2_u = _agg_pass(h_user, h_item, sui, dui, siu, diu, zrow)
    out_item, out_user = _mm_lin(
        agg2_i, cnt_i, h_item, W2_ui_l, b2_ui_l.reshape(1, D), W2_ui_r,
        agg2_u, cnt_u, h_user, W2_iu_l, b2_iu_l.reshape(1, D), W2_iu_r)

    return (out_user, out_item)


# speedup vs baseline: 9.5628x; 1.0208x over previous
import functools

import jax
import jax.numpy as jnp
from jax import lax
from jax.experimental import pallas as pl
from jax.experimental.pallas import tpu as pltpu
from jax.experimental.pallas import tpu_sc as plsc

N = 10000
E = 320000
D = 128
NS = 16
EPT = 20480
E_PAD = NS * EPT
RPT = 632
N_PAD = NS * RPT

A_CHUNK = 64
A_G = 32
A_NGROUP = EPT // (A_G * A_CHUNK)
NBUF = 4

_f32 = jnp.float32


def _wait_idx(e_ref, s, g, buf, sem):
    pltpu.make_async_copy(e_ref.at[s, pl.ds(g * A_G, A_G)], buf, sem).wait()


def _prefetch_idx(e_ref, s, g, buf, sem):
    pltpu.async_copy(e_ref.at[s, pl.ds(g * A_G, A_G)], buf, sem)


def _cnt_loop(s, dst_e, didx, ones_v, acc_s, ssems, isem):
    _prefetch_idx(dst_e, s, 0, didx.at[0], isem)

    def group_body(g, carry):
        par = lax.rem(g, 2)
        dg = didx.at[par]
        _wait_idx(dst_e, s, g, dg, isem)
        _prefetch_idx(dst_e, s, lax.min(g + 1, A_NGROUP - 1),
                      didx.at[1 - par], isem)
        sd = [
            pltpu.async_copy(ones_v, acc_s.at[dg.at[jj]],
                             ssems[jj % NBUF], add=True)
            for jj in range(A_G)
        ]
        for d in sd:
            d.wait()
        return carry

    lax.fori_loop(0, A_NGROUP, group_body, 0)
    _wait_idx(dst_e, s, 0, didx.at[A_NGROUP % 2], isem)


def _agg_loop(s, x_src, src_e, dst_e, sidx, didx, rowbufs, acc_s,
              gsems, ssems, isem0, isem1):
    _prefetch_idx(src_e, s, 0, sidx.at[0], isem0)
    _prefetch_idx(dst_e, s, 0, didx.at[0], isem1)

    def group_body(g, carry):
        par = lax.rem(g, 2)
        sg = sidx.at[par]
        dg = didx.at[par]
        _wait_idx(src_e, s, g, sg, isem0)
        _wait_idx(dst_e, s, g, dg, isem1)
        gn = lax.min(g + 1, A_NGROUP - 1)
        _prefetch_idx(src_e, s, gn, sidx.at[1 - par], isem0)
        _prefetch_idx(dst_e, s, gn, didx.at[1 - par], isem1)

        gd = [None] * A_G
        sd = [None] * A_G
        for jj in range(NBUF - 1):
            gd[jj] = pltpu.async_copy(
                x_src.at[sg.at[jj]], rowbufs[jj % NBUF], gsems[jj % NBUF])
        for jj in range(A_G):
            p = jj % NBUF
            gd[jj].wait()
            sd[jj] = pltpu.async_copy(
                rowbufs[p], acc_s.at[dg.at[jj]], ssems[p], add=True)
            nxt = jj + NBUF - 1
            if nxt < A_G:
                q = nxt % NBUF
                if nxt - NBUF >= 0:
                    sd[nxt - NBUF].wait()
                gd[nxt] = pltpu.async_copy(
                    x_src.at[sg.at[nxt]], rowbufs[q], gsems[q])
        for k in range(max(0, A_G - NBUF), A_G):
            sd[k].wait()
        return carry

    lax.fori_loop(0, A_NGROUP, group_body, 0)
    _wait_idx(src_e, s, 0, sidx.at[A_NGROUP % 2], isem0)
    _wait_idx(dst_e, s, 0, didx.at[A_NGROUP % 2], isem1)


def _split_scratch(rest):
    rowbufs = rest[:NBUF]
    acc_s = rest[NBUF]
    gsems = rest[NBUF + 1:2 * NBUF + 1]
    ssems = rest[2 * NBUF + 1:3 * NBUF + 1]
    isem0 = rest[3 * NBUF + 1]
    isem1 = rest[3 * NBUF + 2]
    return rowbufs, acc_s, gsems, ssems, isem0, isem1


def _cnt_agg_body(xu, xi, sui, dui, siu, diu, zrow, ones_h,
                  agg_i, agg_u, cnt_i, cnt_u,
                  sidx, didx, *rest):
    rowbufs, acc_s, gsems, ssems, isem0, isem1 = _split_scratch(rest)
    c = lax.axis_index("c")
    s = lax.axis_index("s")
    share = pl.ds(s * RPT, RPT)

    pltpu.sync_copy(zrow, acc_s.at[share])
    pltpu.sync_copy(ones_h, rowbufs[0])
    plsc.subcore_barrier()

    def do_cnt(dst_e, cnt_out):
        _cnt_loop(s, dst_e, didx, rowbufs[0], acc_s, ssems, isem1)
        plsc.subcore_barrier()
        pltpu.sync_copy(acc_s.at[share], cnt_out.at[share])
        pltpu.sync_copy(zrow, acc_s.at[share])
        plsc.subcore_barrier()

    @pl.when(c == 0)
    def _():
        do_cnt(dui, cnt_i)

    @pl.when(c == 1)
    def _():
        do_cnt(diu, cnt_u)

    def do_agg(x_src, src_e, dst_e, agg_out):
        _agg_loop(s, x_src, src_e, dst_e, sidx, didx, rowbufs, acc_s,
                  gsems, ssems, isem0, isem1)
        plsc.subcore_barrier()
        pltpu.sync_copy(acc_s.at[share], agg_out.at[share])

    @pl.when(c == 0)
    def _():
        do_agg(xu, sui, dui, agg_i)

    @pl.when(c == 1)
    def _():
        do_agg(xi, siu, diu, agg_u)


def _agg_body(xu, xi, sui, dui, siu, diu, zrow,
              agg_i, agg_u,
              sidx, didx, *rest):
    rowbufs, acc_s, gsems, ssems, isem0, isem1 = _split_scratch(rest)
    c = lax.axis_index("c")
    s = lax.axis_index("s")
    share = pl.ds(s * RPT, RPT)

    pltpu.sync_copy(zrow, acc_s.at[share])
    plsc.subcore_barrier()

    def do_agg(x_src, src_e, dst_e, agg_out):
        _agg_loop(s, x_src, src_e, dst_e, sidx, didx, rowbufs, acc_s,
                  gsems, ssems, isem0, isem1)
        plsc.subcore_barrier()
        pltpu.sync_copy(acc_s.at[share], agg_out.at[share])

    @pl.when(c == 0)
    def _():
        do_agg(xu, sui, dui, agg_i)

    @pl.when(c == 1)
    def _():
        do_agg(xi, siu, diu, agg_u)


def _sc_scratch():
    scratch = [
        pltpu.VMEM((2, A_G, A_CHUNK), jnp.int32),
        pltpu.VMEM((2, A_G, A_CHUNK), jnp.int32),
    ]
    scratch += [pltpu.VMEM((A_CHUNK, D), _f32) for _ in range(NBUF)]
    scratch.append(pltpu.VMEM_SHARED((N_PAD, D), _f32))
    scratch += [pltpu.SemaphoreType.DMA for _ in range(2 * NBUF + 2)]
    return scratch


def _make_cnt_agg():
    t = jax.ShapeDtypeStruct((N_PAD, D), _f32)
    return pl.kernel(
        _cnt_agg_body,
        out_type=(t, t, t, t),
        mesh=plsc.VectorSubcoreMesh(core_axis_name="c", subcore_axis_name="s"),
        scratch_types=_sc_scratch(),
        name="sage_cnt_agg",
    )


def _make_agg():
    t = jax.ShapeDtypeStruct((N_PAD, D), _f32)
    return pl.kernel(
        _agg_body,
        out_type=(t, t),
        mesh=plsc.VectorSubcoreMesh(core_axis_name="c", subcore_axis_name="s"),
        scratch_types=_sc_scratch(),
        name="sage_agg",
    )


_cnt_agg_pass = _make_cnt_agg()
_agg_pass = _make_agg()


def _sage_mm_body(relu, agg_i, cnt_i, xi, Wli, bli, Wri,
                  agg_u, cnt_u, xu, Wlu, blu, Wru, hi, hu):
    def one(agg, cnt, x, Wl, b, Wr, out):
        deg = jnp.maximum(cnt[0:N, 0:1], 1.0)
        mean = agg[0:N, :] / deg
        h = (jnp.dot(mean, Wl[...], preferred_element_type=_f32)
             + b[...]
             + jnp.dot(x[...], Wr[...], preferred_element_type=_f32))
        out[...] = jnp.maximum(h, 0.0) if relu else h

    one(agg_i, cnt_i, xi, Wli, bli, Wri, hi)
    one(agg_u, cnt_u, xu, Wlu, blu, Wru, hu)


def _make_mm(relu):
    return pl.pallas_call(
        functools.partial(_sage_mm_body, relu),
        out_shape=(jax.ShapeDtypeStruct((N, D), _f32),
                   jax.ShapeDtypeStruct((N, D), _f32)),
        name="sage_mm_relu" if relu else "sage_mm",
    )


_mm_relu = _make_mm(True)
_mm_lin = _make_mm(False)


def _prep_edges(e):
    pad = E_PAD - E
    pad_src = jnp.arange(pad, dtype=jnp.int32) % N
    pad_dst = N + (jnp.arange(pad, dtype=jnp.int32) % (N_PAD - N))
    src = jnp.concatenate([e[0].astype(jnp.int32), pad_src])
    dst = jnp.concatenate([e[1].astype(jnp.int32), pad_dst])
    sa = src.reshape(NS, A_G * A_NGROUP, A_CHUNK)
    da = dst.reshape(NS, A_G * A_NGROUP, A_CHUNK)
    return sa, da


def kernel(x_user, x_item, edge_index_ui, edge_index_iu,
           W1_ui_l, b1_ui_l, W1_ui_r, W1_iu_l, b1_iu_l, W1_iu_r,
           W2_ui_l, b2_ui_l, W2_ui_r, W2_iu_l, b2_iu_l, W2_iu_r):
    sui, dui = _prep_edges(edge_index_ui)
    siu, diu = _prep_edges(edge_index_iu)
    zrow = jnp.zeros((RPT, D), _f32)
    ones_h = jnp.ones((A_CHUNK, D), _f32)

    agg1_i, agg1_u, cnt_i, cnt_u = _cnt_agg_pass(
        x_user, x_item, sui, dui, siu, diu, zrow, ones_h)
    h_item, h_user = _mm_relu(
        agg1_i, cnt_i, x_item, W1_ui_l, b1_ui_l.reshape(1, D), W1_ui_r,
        agg1_u, cnt_u, x_user, W1_iu_l, b1_iu_l.reshape(1, D), W1_iu_r)

    agg2_i, agg2_u = _agg_pass(h_user, h_item, sui, dui, siu, diu, zrow)
    out_item, out_user = _mm_lin(
        agg2_i, cnt_i, h_item, W2_ui_l, b2_ui_l.reshape(1, D), W2_ui_r,
        agg2_u, cnt_u, h_user, W2_iu_l, b2_iu_l.reshape(1, D), W2_iu_r)

    return (out_user, out_item)


# speedup vs baseline: 9.6035x; 1.0043x over previous
import functools

import jax
import jax.numpy as jnp
from jax import lax
from jax.experimental import pallas as pl
from jax.experimental.pallas import tpu as pltpu
from jax.experimental.pallas import tpu_sc as plsc

N = 10000
E = 320000
D = 128
NS = 16
EPT = 20480
E_PAD = NS * EPT
RPT = 632
N_PAD = NS * RPT

A_CHUNK = 64
A_G = 16
A_NGROUP = EPT // (A_G * A_CHUNK)
NBUF = 5

_f32 = jnp.float32


def _wait_idx(e_ref, s, g, buf, sem):
    pltpu.make_async_copy(e_ref.at[s, pl.ds(g * A_G, A_G)], buf, sem).wait()


def _prefetch_idx(e_ref, s, g, buf, sem):
    pltpu.async_copy(e_ref.at[s, pl.ds(g * A_G, A_G)], buf, sem)


def _cnt_loop(s, dst_e, didx, ones_v, acc_s, ssems, isem):
    _prefetch_idx(dst_e, s, 0, didx.at[0], isem)

    def group_body(g, carry):
        par = lax.rem(g, 2)
        dg = didx.at[par]
        _wait_idx(dst_e, s, g, dg, isem)
        _prefetch_idx(dst_e, s, lax.min(g + 1, A_NGROUP - 1),
                      didx.at[1 - par], isem)
        sd = [
            pltpu.async_copy(ones_v, acc_s.at[dg.at[jj]],
                             ssems[jj % NBUF], add=True)
            for jj in range(A_G)
        ]
        for d in sd:
            d.wait()
        return carry

    lax.fori_loop(0, A_NGROUP, group_body, 0)
    _wait_idx(dst_e, s, 0, didx.at[A_NGROUP % 2], isem)


def _agg_loop(s, x_src, src_e, dst_e, sidx, didx, rowbufs, acc_s,
              gsems, ssems, isem0, isem1):
    _prefetch_idx(src_e, s, 0, sidx.at[0], isem0)
    _prefetch_idx(dst_e, s, 0, didx.at[0], isem1)

    def group_body(g, carry):
        par = lax.rem(g, 2)
        sg = sidx.at[par]
        dg = didx.at[par]
        _wait_idx(src_e, s, g, sg, isem0)
        _wait_idx(dst_e, s, g, dg, isem1)
        gn = lax.min(g + 1, A_NGROUP - 1)
        _prefetch_idx(src_e, s, gn, sidx.at[1 - par], isem0)
        _prefetch_idx(dst_e, s, gn, didx.at[1 - par], isem1)

        gd = [None] * A_G
        sd = [None] * A_G
        for jj in range(NBUF - 1):
            gd[jj] = pltpu.async_copy(
                x_src.at[sg.at[jj]], rowbufs[jj % NBUF], gsems[jj % NBUF])
        for jj in range(A_G):
            p = jj % NBUF
            gd[jj].wait()
            sd[jj] = pltpu.async_copy(
                rowbufs[p], acc_s.at[dg.at[jj]], ssems[p], add=True)
            nxt = jj + NBUF - 1
            if nxt < A_G:
                q = nxt % NBUF
                if nxt - NBUF >= 0:
                    sd[nxt - NBUF].wait()
                gd[nxt] = pltpu.async_copy(
                    x_src.at[sg.at[nxt]], rowbufs[q], gsems[q])
        for k in range(max(0, A_G - NBUF), A_G):
            sd[k].wait()
        return carry

    lax.fori_loop(0, A_NGROUP, group_body, 0)
    _wait_idx(src_e, s, 0, sidx.at[A_NGROUP % 2], isem0)
    _wait_idx(dst_e, s, 0, didx.at[A_NGROUP % 2], isem1)


def _split_scratch(rest):
    rowbufs = rest[:NBUF]
    acc_s = rest[NBUF]
    gsems = rest[NBUF + 1:2 * NBUF + 1]
    ssems = rest[2 * NBUF + 1:3 * NBUF + 1]
    isem0 = rest[3 * NBUF + 1]
    isem1 = rest[3 * NBUF + 2]
    return rowbufs, acc_s, gsems, ssems, isem0, isem1


def _cnt_agg_body(xu, xi, sui, dui, siu, diu, zrow, ones_h,
                  agg_i, agg_u, cnt_i, cnt_u,
                  sidx, didx, *rest):
    rowbufs, acc_s, gsems, ssems, isem0, isem1 = _split_scratch(rest)
    c = lax.axis_index("c")
    s = lax.axis_index("s")
    share = pl.ds(s * RPT, RPT)

    pltpu.sync_copy(zrow, acc_s.at[share])
    pltpu.sync_copy(ones_h, rowbufs[0])
    plsc.subcore_barrier()

    def do_cnt(dst_e, cnt_out):
        _cnt_loop(s, dst_e, didx, rowbufs[0], acc_s, ssems, isem1)
        plsc.subcore_barrier()
        pltpu.sync_copy(acc_s.at[share], cnt_out.at[share])
        pltpu.sync_copy(zrow, acc_s.at[share])
        plsc.subcore_barrier()

    @pl.when(c == 0)
    def _():
        do_cnt(dui, cnt_i)

    @pl.when(c == 1)
    def _():
        do_cnt(diu, cnt_u)

    def do_agg(x_src, src_e, dst_e, agg_out):
        _agg_loop(s, x_src, src_e, dst_e, sidx, didx, rowbufs, acc_s,
                  gsems, ssems, isem0, isem1)
        plsc.subcore_barrier()
        pltpu.sync_copy(acc_s.at[share], agg_out.at[share])

    @pl.when(c == 0)
    def _():
        do_agg(xu, sui, dui, agg_i)

    @pl.when(c == 1)
    def _():
        do_agg(xi, siu, diu, agg_u)


def _agg_body(xu, xi, sui, dui, siu, diu, zrow,
              agg_i, agg_u,
              sidx, didx, *rest):
    rowbufs, acc_s, gsems, ssems, isem0, isem1 = _split_scratch(rest)
    c = lax.axis_index("c")
    s = lax.axis_index("s")
    share = pl.ds(s * RPT, RPT)

    pltpu.sync_copy(zrow, acc_s.at[share])
    plsc.subcore_barrier()

    def do_agg(x_src, src_e, dst_e, agg_out):
        _agg_loop(s, x_src, src_e, dst_e, sidx, didx, rowbufs, acc_s,
                  gsems, ssems, isem0, isem1)
        plsc.subcore_barrier()
        pltpu.sync_copy(acc_s.at[share], agg_out.at[share])

    @pl.when(c == 0)
    def _():
        do_agg(xu, sui, dui, agg_i)

    @pl.when(c == 1)
    def _():
        do_agg(xi, siu, diu, agg_u)


def _sc_scratch():
    scratch = [
        pltpu.VMEM((2, A_G, A_CHUNK), jnp.int32),
        pltpu.VMEM((2, A_G, A_CHUNK), jnp.int32),
    ]
    scratch += [pltpu.VMEM((A_CHUNK, D), _f32) for _ in range(NBUF)]
    scratch.append(pltpu.VMEM_SHARED((N_PAD, D), _f32))
    scratch += [pltpu.SemaphoreType.DMA for _ in range(2 * NBUF + 2)]
    return scratch


def _make_cnt_agg():
    t = jax.ShapeDtypeStruct((N_PAD, D), _f32)
    return pl.kernel(
        _cnt_agg_body,
        out_type=(t, t, t, t),
        mesh=plsc.VectorSubcoreMesh(core_axis_name="c", subcore_axis_name="s"),
        scratch_types=_sc_scratch(),
        name="sage_cnt_agg",
    )


def _make_agg():
    t = jax.ShapeDtypeStruct((N_PAD, D), _f32)
    return pl.kernel(
        _agg_body,
        out_type=(t, t),
        mesh=plsc.VectorSubcoreMesh(core_axis_name="c", subcore_axis_name="s"),
        scratch_types=_sc_scratch(),
        name="sage_agg",
    )


_cnt_agg_pass = _make_cnt_agg()
_agg_pass = _make_agg()


def _sage_mm_body(relu, agg_i, cnt_i, xi, Wli, bli, Wri,
                  agg_u, cnt_u, xu, Wlu, blu, Wru, hi, hu):
    def one(agg, cnt, x, Wl, b, Wr, out):
        deg = jnp.maximum(cnt[0:N, 0:1], 1.0)
        mean = agg[0:N, :] / deg
        h = (jnp.dot(mean, Wl[...], preferred_element_type=_f32)
             + b[...]
             + jnp.dot(x[...], Wr[...], preferred_element_type=_f32))
        out[...] = jnp.maximum(h, 0.0) if relu else h

    one(agg_i, cnt_i, xi, Wli, bli, Wri, hi)
    one(agg_u, cnt_u, xu, Wlu, blu, Wru, hu)


def _make_mm(relu):
    return pl.pallas_call(
        functools.partial(_sage_mm_body, relu),
        out_shape=(jax.ShapeDtypeStruct((N, D), _f32),
                   jax.ShapeDtypeStruct((N, D), _f32)),
        name="sage_mm_relu" if relu else "sage_mm",
    )


_mm_relu = _make_mm(True)
_mm_lin = _make_mm(False)


def _prep_edges(e):
    pad = E_PAD - E
    pad_src = jnp.arange(pad, dtype=jnp.int32) % N
    pad_dst = N + (jnp.arange(pad, dtype=jnp.int32) % (N_PAD - N))
    src = jnp.concatenate([e[0].astype(jnp.int32), pad_src])
    dst = jnp.concatenate([e[1].astype(jnp.int32), pad_dst])
    sa = src.reshape(NS, A_G * A_NGROUP, A_CHUNK)
    da = dst.reshape(NS, A_G * A_NGROUP, A_CHUNK)
    return sa, da


def kernel(x_user, x_item, edge_index_ui, edge_index_iu,
           W1_ui_l, b1_ui_l, W1_ui_r, W1_iu_l, b1_iu_l, W1_iu_r,
           W2_ui_l, b2_ui_l, W2_ui_r, W2_iu_l, b2_iu_l, W2_iu_r):
    sui, dui = _prep_edges(edge_index_ui)
    siu, diu = _prep_edges(edge_index_iu)
    zrow = jnp.zeros((RPT, D), _f32)
    ones_h = jnp.ones((A_CHUNK, D), _f32)

    agg1_i, agg1_u, cnt_i, cnt_u = _cnt_agg_pass(
        x_user, x_item, sui, dui, siu, diu, zrow, ones_h)
    h_item, h_user = _mm_relu(
        agg1_i, cnt_i, x_item, W1_ui_l, b1_ui_l.reshape(1, D), W1_ui_r,
        agg1_u, cnt_u, x_user, W1_iu_l, b1_iu_l.reshape(1, D), W1_iu_r)

    agg2_i, agg2_u = _agg_pass(h_user, h_item, sui, dui, siu, diu, zrow)
    out_item, out_user = _mm_lin(
        agg2_i, cnt_i, h_item, W2_ui_l, b2_ui_l.reshape(1, D), W2_ui_r,
        agg2_u, cnt_u, h_user, W2_iu_l, b2_iu_l.reshape(1, D), W2_iu_r)

    return (out_user, out_item)
